# Initial kernel scaffold; baseline (speedup 1.0000x reference)
import jax
import jax.numpy as jnp
from jax.experimental import pallas as pl


def kernel(memory, kb_len, dialog_len, dialog_hidden, query, C):
    raise NotImplementedError("write your pallas kernel here")


# speedup vs baseline: 2.8157x; 2.8157x over previous
import functools

import jax
import jax.numpy as jnp
from jax import lax
from jax.experimental import pallas as pl
from jax.experimental.pallas import tpu as pltpu
from jax.experimental.pallas import tpu_sc as plsc

VOCAB = 100000
D = 128
HOPS = 3
B, M, L, S = 128, 1024, 4, 200

NW = 32
CHUNK_O = 32
CHUNK_I = CHUNK_O * L
ROWS_PER_W = (B * M) // NW
N_CHUNKS = ROWS_PER_W // CHUNK_O


def _sc_embed(mem_flat, c2):
    mesh = plsc.VectorSubcoreMesh(core_axis_name="c", subcore_axis_name="s")

    @functools.partial(
        pl.kernel,
        out_type=jax.ShapeDtypeStruct((B, HOPS + 1, M, D), jnp.float32),
        mesh=mesh,
        scratch_types=[
            pltpu.VMEM((CHUNK_I,), jnp.int32),
            pltpu.VMEM((CHUNK_I,), jnp.int32),
            pltpu.VMEM((CHUNK_I, D), jnp.float32),
            pltpu.VMEM((CHUNK_O, D), jnp.float32),
            pltpu.SemaphoreType.DMA,
        ],
    )
    def k(mem_hbm, c_hbm, out_hbm, idx_v, idxk_v, rows_v, out_v, sem):
        wid = lax.axis_index("s") * 2 + lax.axis_index("c")

        def chunk_body(ci, carry):
            out_row = wid * ROWS_PER_W + ci * CHUNK_O
            b = out_row // M
            m = lax.rem(out_row, M)
            pltpu.sync_copy(mem_hbm.at[pl.ds(out_row * L, CHUNK_I)], idx_v)
            for tk in range(HOPS + 1):
                if tk == 0:
                    src_idx = idx_v
                else:
                    def addoff(i, c2_, tk=tk):
                        sl = pl.ds(i * 16, 16)
                        idxk_v[sl] = idx_v[sl] + (tk * VOCAB)
                        return c2_
                    lax.fori_loop(0, CHUNK_I // 16, addoff, 0, unroll=4)
                    src_idx = idxk_v
                pltpu.async_copy(c_hbm.at[src_idx], rows_v, sem).wait()

                def sumrow(i, c2_):
                    for g in range(D // 16):
                        sl = pl.ds(g * 16, 16)
                        out_v[i, sl] = (rows_v[L * i, sl] + rows_v[L * i + 1, sl]
                                        + rows_v[L * i + 2, sl] + rows_v[L * i + 3, sl])
                    return c2_
                lax.fori_loop(0, CHUNK_O, sumrow, 0)
                pltpu.sync_copy(out_v, out_hbm.at[b, tk, pl.ds(m, CHUNK_O)])
            return carry

        lax.fori_loop(0, N_CHUNKS, chunk_body, 0)

    return k(mem_flat, c2)


def _tc_body(kb_ref, dl_ref, e_ref, dh_ref, q_ref, p2_ref, l2_ref):
    bi = pl.program_id(0)
    kbv = kb_ref[bi]
    dlv = dl_ref[bi]
    rows = lax.broadcasted_iota(jnp.int32, (M, S), 0)
    cols = lax.broadcasted_iota(jnp.int32, (M, S), 1)
    sel = jnp.where((rows == kbv + cols) & (cols < dlv), 1.0, 0.0).astype(jnp.float32)
    h_full = jnp.dot(sel, dh_ref[0], preferred_element_type=jnp.float32)
    es = [e_ref[0, tk] + h_full for tk in range(HOPS + 1)]

    q = q_ref[0]
    dn = (((1,), (1,)), ((), ()))
    u = q
    logits = None
    for tk in range(HOPS):
        logits = lax.dot_general(u, es[tk], dn, preferred_element_type=jnp.float32)
        p = jax.nn.softmax(logits, axis=-1)
        u = u + jnp.dot(p, es[tk + 1], preferred_element_type=jnp.float32)
    gp = jax.nn.sigmoid(logits)

    u2 = q
    p2 = None
    l2 = None
    for tk in range(HOPS):
        l2 = lax.dot_general(u2, es[tk], dn, preferred_element_type=jnp.float32) * gp
        p2 = jax.nn.softmax(l2, axis=-1)
        u2 = u2 + jnp.dot(p2 * gp, es[tk + 1], preferred_element_type=jnp.float32)
    p2_ref[0] = p2
    l2_ref[0] = l2


def _tc_hops(kb, dl, e, dh, q, interpret=False):
    return pl.pallas_call(
        _tc_body,
        grid=(B,),
        in_specs=[
            pl.BlockSpec(memory_space=pltpu.SMEM),
            pl.BlockSpec(memory_space=pltpu.SMEM),
            pl.BlockSpec((1, HOPS + 1, M, D), lambda b: (b, 0, 0, 0)),
            pl.BlockSpec((1, S, D), lambda b: (b, 0, 0)),
            pl.BlockSpec((1, 1, D), lambda b: (b, 0, 0)),
        ],
        out_specs=[pl.BlockSpec((1, 1, M), lambda b: (b, 0, 0)),
                   pl.BlockSpec((1, 1, M), lambda b: (b, 0, 0))],
        out_shape=[jax.ShapeDtypeStruct((B, 1, M), jnp.float32),
                   jax.ShapeDtypeStruct((B, 1, M), jnp.float32)],
        interpret=interpret,
    )(kb, dl, e, dh, q.reshape(B, 1, D))


def kernel(memory, kb_len, dialog_len, dialog_hidden, query, C):
    mem_flat = memory.reshape(-1).astype(jnp.int32)
    c2 = C.reshape((HOPS + 1) * VOCAB, D)
    e = _sc_embed(mem_flat, c2)
    probs2, logits2 = _tc_hops(kb_len.astype(jnp.int32), dialog_len.astype(jnp.int32),
                               e, dialog_hidden, query)
    return (probs2.reshape(B, M), logits2.reshape(B, M))


# speedup vs baseline: 4.4859x; 1.5932x over previous
import functools

import jax
import jax.numpy as jnp
from jax import lax
from jax.experimental import pallas as pl
from jax.experimental.pallas import tpu as pltpu
from jax.experimental.pallas import tpu_sc as plsc

VOCAB = 100000
D = 128
HOPS = 3
B, M, L, S = 128, 1024, 4, 200

NW = 32
CHUNK_O = 32
CHUNK_I = CHUNK_O * L
ROWS_PER_W = (B * M) // NW
N_CHUNKS = ROWS_PER_W // CHUNK_O


NT = HOPS + 1


def _sc_embed(mem_flat, c_tables):
    mesh = plsc.VectorSubcoreMesh(core_axis_name="c", subcore_axis_name="s")

    @functools.partial(
        pl.kernel,
        out_type=jax.ShapeDtypeStruct((B, M, NT, D), jnp.float32),
        mesh=mesh,
        scratch_types=[
            pltpu.VMEM((2, CHUNK_I), jnp.int32),
            pltpu.VMEM((2, CHUNK_I, D), jnp.float32),
            pltpu.VMEM((2, CHUNK_O, NT, D), jnp.float32),
            pltpu.SemaphoreType.DMA,
            pltpu.SemaphoreType.DMA,
            pltpu.SemaphoreType.DMA,
            pltpu.SemaphoreType.DMA,
            pltpu.SemaphoreType.DMA,
            pltpu.SemaphoreType.DMA,
        ],
    )
    def k(mem_hbm, c_hbm, out_hbm, idx_v, rows_v, out_v,
          isem0, isem1, gsem0, gsem1, osem0, osem1):
        wid = lax.axis_index("s") * 2 + lax.axis_index("c")
        row0 = wid * ROWS_PER_W
        isems = (isem0, isem1)
        gsems = (gsem0, gsem1)
        osems = (osem0, osem1)

        def idx_load(ci, buf, sem):
            return pltpu.async_copy(
                mem_hbm.at[pl.ds((row0 + ci * CHUNK_O) * L, CHUNK_I)],
                idx_v.at[buf], sem)

        def gather(cbuf, tk, gbuf):
            return pltpu.async_copy(c_hbm.at[tk].at[idx_v.at[cbuf]],
                                    rows_v.at[gbuf], gsems[gbuf])

        def wait_gather(gbuf):
            pltpu.make_async_copy(c_hbm.at[0].at[idx_v.at[0]],
                                  rows_v.at[gbuf], gsems[gbuf]).wait()

        idx_load(0, 0, isems[0]).wait()
        gather(0, 0, 0)

        def pair_body(cj, carry):
            for par in range(2):
                ci = 2 * cj + par
                out_row = row0 + ci * CHUNK_O
                b = out_row // M
                m = lax.rem(out_row, M)
                obuf = par
                cbuf = par

                @pl.when(ci + 1 < N_CHUNKS)
                def _(ci=ci, cbuf=cbuf):
                    idx_load(ci + 1, 1 - cbuf, isems[1 - cbuf])

                @pl.when(ci >= 2)
                def _(obuf=obuf):
                    pltpu.make_async_copy(out_v.at[obuf],
                                          out_hbm.at[0, pl.ds(0, CHUNK_O)],
                                          osems[obuf]).wait()

                for tk in range(NT):
                    gbuf = tk % 2
                    if tk < NT - 1:
                        gather(cbuf, tk + 1, 1 - gbuf)
                    else:
                        @pl.when(ci + 1 < N_CHUNKS)
                        def _(cbuf=cbuf, gbuf=gbuf):
                            pltpu.make_async_copy(
                                mem_hbm.at[pl.ds(0, CHUNK_I)],
                                idx_v.at[1 - cbuf], isems[1 - cbuf]).wait()
                            gather(1 - cbuf, 0, 1 - gbuf)
                    wait_gather(gbuf)

                    def sumrow(i, acc, gbuf=gbuf, tk=tk, obuf=obuf):
                        for g in range(D // 16):
                            sl = pl.ds(g * 16, 16)
                            out_v[obuf, i, tk, sl] = (
                                rows_v[gbuf, L * i, sl] + rows_v[gbuf, L * i + 1, sl]
                                + rows_v[gbuf, L * i + 2, sl] + rows_v[gbuf, L * i + 3, sl])
                        return acc
                    lax.fori_loop(0, CHUNK_O, sumrow, 0, unroll=2)

                pltpu.async_copy(out_v.at[obuf], out_hbm.at[b, pl.ds(m, CHUNK_O)],
                                 osems[obuf])
            return carry

        lax.fori_loop(0, N_CHUNKS // 2, pair_body, 0)

        for obuf in range(2):
            pltpu.make_async_copy(out_v.at[obuf],
                                  out_hbm.at[0, pl.ds(0, CHUNK_O)],
                                  osems[obuf]).wait()

    return k(mem_flat, c_tables)


def _tc_body(kb_ref, dl_ref, e_ref, dh_ref, q_ref, p2_ref, l2_ref):
    bi = pl.program_id(0)
    kbv = kb_ref[bi]
    dlv = dl_ref[bi]
    rows = lax.broadcasted_iota(jnp.int32, (M, S), 0)
    cols = lax.broadcasted_iota(jnp.int32, (M, S), 1)
    sel = jnp.where((rows == kbv + cols) & (cols < dlv), 1.0, 0.0).astype(jnp.float32)
    h_full = jnp.dot(sel, dh_ref[0], preferred_element_type=jnp.float32)
    e_all = e_ref[0]
    es = [e_all[:, tk * D:(tk + 1) * D] + h_full for tk in range(HOPS + 1)]

    q = q_ref[0]
    dn = (((1,), (1,)), ((), ()))
    u = q
    logits = None
    for tk in range(HOPS):
        logits = lax.dot_general(u, es[tk], dn, preferred_element_type=jnp.float32)
        p = jax.nn.softmax(logits, axis=-1)
        u = u + jnp.dot(p, es[tk + 1], preferred_element_type=jnp.float32)
    gp = jax.nn.sigmoid(logits)

    u2 = q
    p2 = None
    l2 = None
    for tk in range(HOPS):
        l2 = lax.dot_general(u2, es[tk], dn, preferred_element_type=jnp.float32) * gp
        p2 = jax.nn.softmax(l2, axis=-1)
        u2 = u2 + jnp.dot(p2 * gp, es[tk + 1], preferred_element_type=jnp.float32)
    p2_ref[0] = p2
    l2_ref[0] = l2


def _tc_hops(kb, dl, e, dh, q, interpret=False):
    return pl.pallas_call(
        _tc_body,
        grid=(B,),
        in_specs=[
            pl.BlockSpec(memory_space=pltpu.SMEM),
            pl.BlockSpec(memory_space=pltpu.SMEM),
            pl.BlockSpec((1, M, NT * D), lambda b: (b, 0, 0)),
            pl.BlockSpec((1, S, D), lambda b: (b, 0, 0)),
            pl.BlockSpec((1, 1, D), lambda b: (b, 0, 0)),
        ],
        out_specs=[pl.BlockSpec((1, 1, M), lambda b: (b, 0, 0)),
                   pl.BlockSpec((1, 1, M), lambda b: (b, 0, 0))],
        out_shape=[jax.ShapeDtypeStruct((B, 1, M), jnp.float32),
                   jax.ShapeDtypeStruct((B, 1, M), jnp.float32)],
        interpret=interpret,
    )(kb, dl, e, dh, q.reshape(B, 1, D))


def kernel(memory, kb_len, dialog_len, dialog_hidden, query, C):
    mem_flat = memory.reshape(-1).astype(jnp.int32)
    e = _sc_embed(mem_flat, C)
    probs2, logits2 = _tc_hops(kb_len.astype(jnp.int32), dialog_len.astype(jnp.int32),
                               e.reshape(B, M, NT * D), dialog_hidden, query)
    return (probs2.reshape(B, M), logits2.reshape(B, M))
